# baseline (speedup 1.0000x reference)
import functools
import math

import jax
import jax.numpy as jnp
from jax import lax
from jax.experimental import pallas as pl
from jax.experimental.pallas import tpu as pltpu
from jax.experimental.pallas import tpu_sc as plsc

_DT_SCALE = float(math.log1p(168.0))
_NUM_CORES = 2
_NUM_SUBCORES = 16
_NW = _NUM_CORES * _NUM_SUBCORES
_LANES = 16


@functools.lru_cache(maxsize=None)
def _build_sc_kernel(n, chunk):
    per_worker = n // _NW
    n_chunks = per_worker // chunk
    mesh = plsc.VectorSubcoreMesh(
        core_axis_name="c", subcore_axis_name="s",
        num_cores=_NUM_CORES, num_subcores=_NUM_SUBCORES)

    @functools.partial(
        pl.kernel,
        out_type=jax.ShapeDtypeStruct((n,), jnp.float32),
        mesh=mesh,
        compiler_params=pltpu.CompilerParams(needs_layout_passes=False),
        scratch_types=[
            pltpu.VMEM((_LANES,), jnp.float32),
            pltpu.VMEM((_LANES,), jnp.float32),
            pltpu.VMEM((chunk,), jnp.float32),
            pltpu.VMEM((chunk,), jnp.float32),
            pltpu.VMEM((chunk,), jnp.int32),
            pltpu.VMEM((chunk,), jnp.float32),
        ],
    )
    def sc_kernel(mpc_hbm, dt_hbm, idx_hbm, ltau_hbm, out_hbm,
                  ltau_v, tbl_v, mpc_v, dt_v, idx_v, out_v):
        wid = lax.axis_index("s") * _NUM_CORES + lax.axis_index("c")
        pltpu.sync_copy(ltau_hbm, ltau_v)
        tbl_v[...] = -jnp.exp(-ltau_v[...])
        base = wid * per_worker

        def chunk_body(c, carry):
            off = base + c * chunk
            pltpu.sync_copy(mpc_hbm.at[pl.ds(off, chunk)], mpc_v)
            pltpu.sync_copy(dt_hbm.at[pl.ds(off, chunk)], dt_v)
            pltpu.sync_copy(idx_hbm.at[pl.ds(off, chunk)], idx_v)

            def vec_body(i, carry2):
                s = pl.ds(pl.multiple_of(i * _LANES, _LANES), _LANES)
                g = plsc.load_gather(tbl_v, [idx_v[s]])
                e = jnp.exp(dt_v[s] * _DT_SCALE)
                decay = jnp.exp((e - 1.0) * g)
                out_v[s] = 1.0 - (1.0 - mpc_v[s]) * decay
                return carry2

            lax.fori_loop(0, chunk // _LANES, vec_body, 0, unroll=4)
            pltpu.sync_copy(out_v, out_hbm.at[pl.ds(off, chunk)])
            return carry

        lax.fori_loop(0, n_chunks, chunk_body, 0)

    return sc_kernel


def kernel(mpc, delta_t, muscle_idx, log_tau):
    n = mpc.shape[0]
    idx = muscle_idx.astype(jnp.int32)
    ltau16 = jnp.zeros((_LANES,), jnp.float32).at[:log_tau.shape[0]].set(
        log_tau.astype(jnp.float32))
    return _build_sc_kernel(n, 10240)(mpc, delta_t, idx, ltau16)


# speedup vs baseline: 4.3579x; 4.3579x over previous
import functools
import math

import jax
import jax.numpy as jnp
from jax import lax
from jax.experimental import pallas as pl
from jax.experimental.pallas import tpu as pltpu
from jax.experimental.pallas import tpu_sc as plsc

_DT_SCALE = float(math.log1p(168.0))
_NUM_CORES = 2
_NUM_SUBCORES = 16
_NW = _NUM_CORES * _NUM_SUBCORES
_LANES = 16


@functools.lru_cache(maxsize=None)
def _build_sc_kernel(n, chunk, unroll):
    per_worker = n // _NW
    n_chunks = per_worker // chunk
    mesh = plsc.VectorSubcoreMesh(
        core_axis_name="c", subcore_axis_name="s",
        num_cores=_NUM_CORES, num_subcores=_NUM_SUBCORES)

    @functools.partial(
        pl.kernel,
        out_type=jax.ShapeDtypeStruct((n,), jnp.float32),
        mesh=mesh,
        compiler_params=pltpu.CompilerParams(needs_layout_passes=False),
        scratch_types=[
            pltpu.VMEM((_LANES,), jnp.float32),
            pltpu.VMEM((_LANES,), jnp.float32),
            [pltpu.VMEM((chunk,), jnp.float32)] * 2,
            [pltpu.VMEM((chunk,), jnp.float32)] * 2,
            [pltpu.VMEM((chunk,), jnp.int32)] * 2,
            [pltpu.VMEM((chunk,), jnp.float32)] * 2,
            [pltpu.SemaphoreType.DMA] * 2,
            [pltpu.SemaphoreType.DMA] * 2,
        ],
    )
    def sc_kernel(mpc_hbm, dt_hbm, idx_hbm, ltau_hbm, out_hbm,
                  ltau_v, tbl_v, mpc_v, dt_v, idx_v, out_v, in_sem, out_sem):
        wid = lax.axis_index("s") * _NUM_CORES + lax.axis_index("c")
        pltpu.sync_copy(ltau_hbm, ltau_v)
        tbl_v[...] = -jnp.exp(-ltau_v[...])
        base = wid * per_worker

        def start_in(c, b):
            off = base + c * chunk
            return (
                pltpu.async_copy(mpc_hbm.at[pl.ds(off, chunk)], mpc_v[b], in_sem[b]),
                pltpu.async_copy(dt_hbm.at[pl.ds(off, chunk)], dt_v[b], in_sem[b]),
                pltpu.async_copy(idx_hbm.at[pl.ds(off, chunk)], idx_v[b], in_sem[b]),
            )

        def compute(b):
            @plsc.parallel_loop(0, chunk, step=_LANES, unroll=unroll)
            def _(i):
                s = pl.ds(pl.multiple_of(i, _LANES), _LANES)
                g = plsc.load_gather(tbl_v, [idx_v[b][s]])
                e = jnp.exp(dt_v[b][s] * _DT_SCALE)
                decay = jnp.exp((e - 1.0) * g)
                out_v[b][s] = 1.0 - (1.0 - mpc_v[b][s]) * decay

        in_flight = [None, None]
        out_flight = [None, None]
        in_flight[0] = start_in(0, 0)
        for c in range(n_chunks):
            b = c % 2
            if c + 1 < n_chunks:
                in_flight[1 - b] = start_in(c + 1, 1 - b)
            for d in in_flight[b]:
                d.wait()
            if out_flight[b] is not None:
                out_flight[b].wait()
            compute(b)
            off = base + c * chunk
            out_flight[b] = pltpu.async_copy(
                out_v[b], out_hbm.at[pl.ds(off, chunk)], out_sem[b])
        for b in range(2):
            if out_flight[b] is not None:
                out_flight[b].wait()

    return sc_kernel


def kernel(mpc, delta_t, muscle_idx, log_tau):
    n = mpc.shape[0]
    idx = muscle_idx.astype(jnp.int32)
    ltau16 = jnp.zeros((_LANES,), jnp.float32).at[:log_tau.shape[0]].set(
        log_tau.astype(jnp.float32))
    return _build_sc_kernel(n, 10240, 8)(mpc, delta_t, idx, ltau16)


# speedup vs baseline: 4.4508x; 1.0213x over previous
import functools
import math

import jax
import jax.numpy as jnp
from jax import lax
from jax.experimental import pallas as pl
from jax.experimental.pallas import tpu as pltpu
from jax.experimental.pallas import tpu_sc as plsc

_DT_SCALE = float(math.log1p(168.0))
_NUM_CORES = 2
_NUM_SUBCORES = 16
_NW = _NUM_CORES * _NUM_SUBCORES
_LANES = 16


@functools.lru_cache(maxsize=None)
def _build_sc_kernel(n, chunk, unroll):
    per_worker = n // _NW
    n_chunks = per_worker // chunk
    mesh = plsc.VectorSubcoreMesh(
        core_axis_name="c", subcore_axis_name="s",
        num_cores=_NUM_CORES, num_subcores=_NUM_SUBCORES)

    @functools.partial(
        pl.kernel,
        out_type=jax.ShapeDtypeStruct((n,), jnp.float32),
        mesh=mesh,
        compiler_params=pltpu.CompilerParams(
            needs_layout_passes=False, skip_device_barrier=True),
        scratch_types=[
            pltpu.VMEM((_LANES,), jnp.float32),
            pltpu.VMEM((_LANES,), jnp.float32),
            [pltpu.VMEM((chunk,), jnp.float32)] * 2,
            [pltpu.VMEM((chunk,), jnp.float32)] * 2,
            [pltpu.VMEM((chunk,), jnp.int32)] * 2,
            [pltpu.VMEM((chunk,), jnp.float32)] * 2,
            [pltpu.SemaphoreType.DMA] * 2,
            [pltpu.SemaphoreType.DMA] * 2,
        ],
    )
    def sc_kernel(mpc_hbm, dt_hbm, idx_hbm, ltau_hbm, out_hbm,
                  ltau_v, tbl_v, mpc_v, dt_v, idx_v, out_v, in_sem, out_sem):
        wid = lax.axis_index("s") * _NUM_CORES + lax.axis_index("c")
        nt = 15
        pltpu.sync_copy(ltau_hbm, ltau_v.at[pl.ds(0, nt)])
        tbl_v[...] = -jnp.exp(-ltau_v[...])
        base = wid * per_worker

        def start_in(c, b):
            off = base + c * chunk
            return (
                pltpu.async_copy(mpc_hbm.at[pl.ds(off, chunk)], mpc_v[b], in_sem[b]),
                pltpu.async_copy(dt_hbm.at[pl.ds(off, chunk)], dt_v[b], in_sem[b]),
                pltpu.async_copy(idx_hbm.at[pl.ds(off, chunk)], idx_v[b], in_sem[b]),
            )

        def compute(b):
            @plsc.parallel_loop(0, chunk, step=_LANES, unroll=unroll)
            def _(i):
                s = pl.ds(pl.multiple_of(i, _LANES), _LANES)
                g = plsc.load_gather(tbl_v, [idx_v[b][s]])
                e = jnp.exp(dt_v[b][s] * _DT_SCALE)
                decay = jnp.exp((e - 1.0) * g)
                out_v[b][s] = 1.0 - (1.0 - mpc_v[b][s]) * decay

        in_flight = [None, None]
        out_flight = [None, None]
        in_flight[0] = start_in(0, 0)
        for c in range(n_chunks):
            b = c % 2
            if c + 1 < n_chunks:
                in_flight[1 - b] = start_in(c + 1, 1 - b)
            for d in in_flight[b]:
                d.wait()
            if out_flight[b] is not None:
                out_flight[b].wait()
            compute(b)
            off = base + c * chunk
            out_flight[b] = pltpu.async_copy(
                out_v[b], out_hbm.at[pl.ds(off, chunk)], out_sem[b])
        for b in range(2):
            if out_flight[b] is not None:
                out_flight[b].wait()

    return sc_kernel


def kernel(mpc, delta_t, muscle_idx, log_tau):
    n = mpc.shape[0]
    idx = muscle_idx.astype(jnp.int32)
    return _build_sc_kernel(n, 10240, 8)(mpc, delta_t, idx, log_tau)
